# Initial kernel scaffold; baseline (speedup 1.0000x reference)
import jax
import jax.numpy as jnp
from jax.experimental import pallas as pl


def kernel(embedding, edge_index, W_gcn, b_gcn, W_out, b_out):
    raise NotImplementedError("write your pallas kernel here")


# speedup vs baseline: 25.1760x; 25.1760x over previous
import functools

import jax
import jax.numpy as jnp
from jax import lax
from jax.experimental import pallas as pl
from jax.experimental.pallas import tpu as pltpu, tpu_sc as plsc

N_NODES = 50000
IN_DIM = 64
HID_DIM = 32
HALF = HID_DIM // 2
NUM_CLASS = 10
N_EDGES = 800000

NC = 2
NS = 16
NW = NC * NS
L = 16

CHUNK = 128
EPT = 25088
NCHUNK = EPT // CHUNK
E_PAD = EPT * NW
N_PAD = 50176
STRIPE = N_PAD // NS
PAD_ROWS = N_PAD - N_NODES
BOUNCE = STRIPE // 4


def _deg_body(dst_hbm, deg_out, dst_v, ones_v, zero_v, deg_sh):
    c = lax.axis_index("c")
    s = lax.axis_index("s")
    w = c * NS + s

    for i in range(CHUNK // L):
        ones_v[pl.ds(i * L, L)] = jnp.ones((L,), jnp.float32)

    @pl.loop(0, STRIPE // L)
    def _zero(i):
        zero_v[pl.ds(i * L, L)] = jnp.zeros((L,), jnp.float32)

    pltpu.sync_copy(zero_v, deg_sh.at[pl.ds(s * STRIPE, STRIPE)])
    plsc.subcore_barrier()

    pltpu.sync_copy(dst_hbm.at[w], dst_v)

    @pl.loop(0, NCHUNK)
    def _scatter(j):
        pltpu.sync_copy(ones_v, deg_sh.at[dst_v.at[j]], add=True)

    plsc.subcore_barrier()
    pltpu.sync_copy(deg_sh.at[pl.ds(s * STRIPE, STRIPE)], zero_v)
    pltpu.sync_copy(zero_v, deg_out.at[pl.ds(c * N_PAD + s * STRIPE, STRIPE)])


_deg_kernel = functools.partial(
    pl.kernel,
    out_type=jax.ShapeDtypeStruct((NC * N_PAD,), jnp.float32),
    mesh=plsc.VectorSubcoreMesh(core_axis_name="c", subcore_axis_name="s"),
    compiler_params=pltpu.CompilerParams(use_tc_tiling_on_sc=False),
    scratch_types=[
        pltpu.VMEM((NCHUNK, CHUNK), jnp.int32),
        pltpu.VMEM((CHUNK,), jnp.float32),
        pltpu.VMEM((STRIPE,), jnp.float32),
        pltpu.VMEM_SHARED((N_PAD,), jnp.float32),
    ],
)(_deg_body)


def _edge_body(y_hbm, src_hbm, dst_hbm, part_out, src_v, dst_v, rows_v,
               bnc_v, sem, h_sh):
    c = lax.axis_index("c")
    s = lax.axis_index("s")
    w = c * NS + s

    for k in range(4):
        r = s * STRIPE + k * BOUNCE
        pltpu.sync_copy(y_hbm.at[pl.ds(r, BOUNCE)], bnc_v)
        pltpu.sync_copy(bnc_v, h_sh.at[pl.ds(r, BOUNCE)])

    pltpu.sync_copy(src_hbm.at[w], src_v)
    pltpu.sync_copy(dst_hbm.at[w], dst_v)
    plsc.subcore_barrier()

    @pl.loop(0, NCHUNK)
    def _edges(j):
        pltpu.async_copy(y_hbm.at[src_v.at[j]], rows_v, sem).wait()
        pltpu.sync_copy(rows_v, h_sh.at[dst_v.at[j]], add=True)

    plsc.subcore_barrier()
    for k in range(4):
        r = s * STRIPE + k * BOUNCE
        pltpu.sync_copy(h_sh.at[pl.ds(r, BOUNCE)], bnc_v)
        pltpu.sync_copy(bnc_v, part_out.at[c, pl.ds(r, BOUNCE)])


_edge_kernel = functools.partial(
    pl.kernel,
    out_type=jax.ShapeDtypeStruct((NC, N_PAD, HALF), jnp.float32),
    mesh=plsc.VectorSubcoreMesh(core_axis_name="c", subcore_axis_name="s"),
    compiler_params=pltpu.CompilerParams(use_tc_tiling_on_sc=False),
    scratch_types=[
        pltpu.VMEM((NCHUNK, CHUNK), jnp.int32),
        pltpu.VMEM((NCHUNK, CHUNK), jnp.int32),
        pltpu.VMEM((CHUNK, HALF), jnp.float32),
        pltpu.VMEM((BOUNCE, HALF), jnp.float32),
        pltpu.SemaphoreType.DMA,
        pltpu.VMEM_SHARED((N_PAD, HALF), jnp.float32),
    ],
)(_edge_body)


def _y_body(emb_ref, w_ref, d0_ref, d1_ref, ya_ref, yb_ref):
    deg = d0_ref[...] + d1_ref[...] + 1.0
    dis = lax.rsqrt(deg)
    xw = jnp.dot(emb_ref[...], w_ref[...], preferred_element_type=jnp.float32)
    y = dis * xw
    ya_ref[...] = y[:, :HALF]
    yb_ref[...] = y[:, HALF:]


def _y_call(emb, w_gcn, d0, d1):
    blk = 2000
    grid = (N_NODES // blk,)
    return pl.pallas_call(
        _y_body,
        grid=grid,
        in_specs=[
            pl.BlockSpec((blk, IN_DIM), lambda i: (i, 0)),
            pl.BlockSpec((IN_DIM, HID_DIM), lambda i: (0, 0)),
            pl.BlockSpec((blk, 1), lambda i: (i, 0)),
            pl.BlockSpec((blk, 1), lambda i: (i, 0)),
        ],
        out_specs=[
            pl.BlockSpec((blk, HALF), lambda i: (i, 0)),
            pl.BlockSpec((blk, HALF), lambda i: (i, 0)),
        ],
        out_shape=[
            jax.ShapeDtypeStruct((N_NODES, HALF), jnp.float32),
            jax.ShapeDtypeStruct((N_NODES, HALF), jnp.float32),
        ],
    )(emb, w_gcn, d0, d1)


def _z_body(ya_ref, yb_ref, p0a_ref, p1a_ref, p0b_ref, p1b_ref, d0_ref,
            d1_ref, bg_ref, wo_ref, bo_ref, z_ref):
    deg = d0_ref[...] + d1_ref[...] + 1.0
    dis = lax.rsqrt(deg)
    ha = dis * (p0a_ref[...] + p1a_ref[...] - ya_ref[...])
    hb = dis * (p0b_ref[...] + p1b_ref[...] - yb_ref[...])
    h = jnp.concatenate([ha, hb], axis=1)
    e = jnp.maximum(h + bg_ref[...], 0.0)
    z_ref[...] = (
        jnp.dot(e, wo_ref[...], preferred_element_type=jnp.float32)
        + bo_ref[...]
    )


def _z_call(ya, yb, p0a, p1a, p0b, p1b, d0, d1, b_gcn, w_out_pad, b_out_pad,
            ncls_pad):
    blk = 2000
    grid = (N_NODES // blk,)
    half_spec = pl.BlockSpec((blk, HALF), lambda i: (i, 0))
    col_spec = pl.BlockSpec((blk, 1), lambda i: (i, 0))
    return pl.pallas_call(
        _z_body,
        grid=grid,
        in_specs=[
            half_spec, half_spec, half_spec, half_spec, half_spec, half_spec,
            col_spec, col_spec,
            pl.BlockSpec((1, HID_DIM), lambda i: (0, 0)),
            pl.BlockSpec((HID_DIM, ncls_pad), lambda i: (0, 0)),
            pl.BlockSpec((1, ncls_pad), lambda i: (0, 0)),
        ],
        out_specs=pl.BlockSpec((blk, ncls_pad), lambda i: (i, 0)),
        out_shape=jax.ShapeDtypeStruct((N_NODES, ncls_pad), jnp.float32),
    )(ya, yb, p0a, p1a, p0b, p1b, d0, d1, b_gcn, w_out_pad, b_out_pad)


def kernel(embedding, edge_index, W_gcn, b_gcn, W_out, b_out):
    src = edge_index[0].astype(jnp.int32)
    dst = edge_index[1].astype(jnp.int32)

    npad = E_PAD - N_EDGES
    pad_i = jnp.arange(npad, dtype=jnp.int32)
    pad_src = (pad_i * 131) % N_NODES
    pad_dst = N_NODES + (pad_i % PAD_ROWS)
    srcp = jnp.concatenate([src, pad_src]).reshape(NW, NCHUNK, CHUNK)
    dstp = jnp.concatenate([dst, pad_dst]).reshape(NW, NCHUNK, CHUNK)

    deg_pair = _deg_kernel(dstp).reshape(NC, N_PAD)
    d0 = deg_pair[0, :N_NODES].reshape(N_NODES, 1)
    d1 = deg_pair[1, :N_NODES].reshape(N_NODES, 1)

    ya, yb = _y_call(embedding, W_gcn, d0, d1)
    zpad = jnp.zeros((N_PAD - N_NODES, HALF), jnp.float32)
    ya_pad = jnp.concatenate([ya, zpad], axis=0)
    yb_pad = jnp.concatenate([yb, zpad], axis=0)

    parts_a = _edge_kernel(ya_pad, srcp, dstp)
    parts_b = _edge_kernel(yb_pad, srcp, dstp)

    ncls_pad = 128
    w_out_pad = jnp.zeros((HID_DIM, ncls_pad), jnp.float32).at[:, :NUM_CLASS].set(W_out)
    b_out_pad = jnp.zeros((1, ncls_pad), jnp.float32).at[0, :NUM_CLASS].set(b_out)

    z_pad = _z_call(ya, yb,
                    parts_a[0, :N_NODES], parts_a[1, :N_NODES],
                    parts_b[0, :N_NODES], parts_b[1, :N_NODES],
                    d0, d1, b_gcn.reshape(1, HID_DIM),
                    w_out_pad, b_out_pad, ncls_pad)
    return z_pad[:, :NUM_CLASS]


# speedup vs baseline: 40.7269x; 1.6177x over previous
import functools

import jax
import jax.numpy as jnp
from jax import lax
from jax.experimental import pallas as pl
from jax.experimental.pallas import tpu as pltpu, tpu_sc as plsc

N_NODES = 50000
IN_DIM = 64
HID_DIM = 32
HALF = HID_DIM // 2
NUM_CLASS = 10
N_EDGES = 800000

NC = 2
NS = 16
NW = NC * NS
L = 16

CHUNK = 128
EPT = 25088
NCHUNK = EPT // CHUNK
E_PAD = EPT * NW
N_PAD = 50176
STRIPE = N_PAD // NS
PAD_ROWS = N_PAD - N_NODES
BOUNCE = STRIPE // 4


def _deg_body(dst_hbm, deg_out, dst_v, ones_v, zero_v, deg_sh):
    c = lax.axis_index("c")
    s = lax.axis_index("s")
    w = c * NS + s

    for i in range(CHUNK // L):
        ones_v[pl.ds(i * L, L)] = jnp.ones((L,), jnp.float32)

    @pl.loop(0, STRIPE // L)
    def _zero(i):
        zero_v[pl.ds(i * L, L)] = jnp.zeros((L,), jnp.float32)

    pltpu.sync_copy(zero_v, deg_sh.at[pl.ds(s * STRIPE, STRIPE)])
    plsc.subcore_barrier()

    pltpu.sync_copy(dst_hbm.at[w], dst_v)

    @pl.loop(0, NCHUNK)
    def _scatter(j):
        pltpu.sync_copy(ones_v, deg_sh.at[dst_v.at[j]], add=True)

    plsc.subcore_barrier()
    pltpu.sync_copy(deg_sh.at[pl.ds(s * STRIPE, STRIPE)], zero_v)
    pltpu.sync_copy(zero_v, deg_out.at[pl.ds(c * N_PAD + s * STRIPE, STRIPE)])


_deg_kernel = functools.partial(
    pl.kernel,
    out_type=jax.ShapeDtypeStruct((NC * N_PAD,), jnp.float32),
    mesh=plsc.VectorSubcoreMesh(core_axis_name="c", subcore_axis_name="s"),
    compiler_params=pltpu.CompilerParams(use_tc_tiling_on_sc=False),
    scratch_types=[
        pltpu.VMEM((NCHUNK, CHUNK), jnp.int32),
        pltpu.VMEM((CHUNK,), jnp.float32),
        pltpu.VMEM((STRIPE,), jnp.float32),
        pltpu.VMEM_SHARED((N_PAD,), jnp.float32),
    ],
)(_deg_body)


NCHUNK2 = E_PAD // NS // CHUNK
SLAB = 56
NSLAB = NCHUNK2 // SLAB
NB = 8
LAG = 4


def _edge_body(y_hbm, srca_hbm, srcb_hbm, dst_hbm, part_out,
               src_v, dst_v, r0, r1, r2, r3, r4, r5, r6, r7,
               g0, g1, g2, g3, g4, g5, g6, g7,
               s0, s1, s2, s3, s4, s5, s6, s7,
               bnc_v, h_sh):
    c = lax.axis_index("c")
    s = lax.axis_index("s")
    rows = [r0, r1, r2, r3, r4, r5, r6, r7]
    gsem = [g0, g1, g2, g3, g4, g5, g6, g7]
    ssem = [s0, s1, s2, s3, s4, s5, s6, s7]

    for k in range(4):
        r = s * STRIPE + k * BOUNCE
        pltpu.sync_copy(y_hbm.at[pl.ds(c * N_PAD + r, BOUNCE)], bnc_v)
        pltpu.sync_copy(bnc_v, h_sh.at[pl.ds(r, BOUNCE)])
    plsc.subcore_barrier()

    def g_issue(lj, b):
        pltpu.async_copy(y_hbm.at[src_v.at[lj]], rows[b], gsem[b])

    def g_wait(lj, b):
        pltpu.make_async_copy(y_hbm.at[src_v.at[lj]], rows[b], gsem[b]).wait()

    def s_issue(lj, b):
        pltpu.async_copy(rows[b], h_sh.at[dst_v.at[lj]], ssem[b], add=True)

    def s_wait(lj, b):
        pltpu.make_async_copy(rows[b], h_sh.at[dst_v.at[lj]], ssem[b]).wait()

    @pl.loop(0, NSLAB)
    def _slab(t):
        @pl.when(c == 0)
        def _():
            pltpu.sync_copy(srca_hbm.at[s, pl.ds(t * SLAB, SLAB)], src_v)

        @pl.when(c == 1)
        def _():
            pltpu.sync_copy(srcb_hbm.at[s, pl.ds(t * SLAB, SLAB)], src_v)

        pltpu.sync_copy(dst_hbm.at[s, pl.ds(t * SLAB, SLAB)], dst_v)

        for b in range(NB):
            g_issue(b, b)
        for b in range(LAG):
            g_wait(b, b)
            s_issue(b, b)

        @pl.loop(1, SLAB // NB)
        def _grp(g):
            for b in range(NB):
                lj = g * NB + b
                s_wait(lj - NB, b)
                g_issue(lj, b)
                bb = (b + LAG) % NB
                g_wait(lj - LAG, bb)
                s_issue(lj - LAG, bb)

        last = SLAB - NB
        for b in range(LAG):
            g_wait(last + LAG + b, LAG + b)
            s_issue(last + LAG + b, LAG + b)
        for b in range(NB):
            s_wait(last + b, b)

    plsc.subcore_barrier()
    for k in range(4):
        r = s * STRIPE + k * BOUNCE
        pltpu.sync_copy(h_sh.at[pl.ds(r, BOUNCE)], bnc_v)
        pltpu.sync_copy(bnc_v, part_out.at[c, pl.ds(r, BOUNCE)])


_edge_kernel = functools.partial(
    pl.kernel,
    out_type=jax.ShapeDtypeStruct((NC, N_PAD, HALF), jnp.float32),
    mesh=plsc.VectorSubcoreMesh(core_axis_name="c", subcore_axis_name="s"),
    compiler_params=pltpu.CompilerParams(use_tc_tiling_on_sc=False),
    scratch_types=(
        [
            pltpu.VMEM((SLAB, CHUNK), jnp.int32),
            pltpu.VMEM((SLAB, CHUNK), jnp.int32),
        ]
        + [pltpu.VMEM((CHUNK, HALF), jnp.float32) for _ in range(NB)]
        + [pltpu.SemaphoreType.DMA for _ in range(2 * NB)]
        + [
            pltpu.VMEM((BOUNCE, HALF), jnp.float32),
            pltpu.VMEM_SHARED((N_PAD, HALF), jnp.float32),
        ]
    ),
)(_edge_body)


def _y_body(emb_ref, w_ref, d0_ref, d1_ref, ya_ref, yb_ref):
    deg = d0_ref[...] + d1_ref[...] + 1.0
    dis = lax.rsqrt(deg)
    xw = jnp.dot(emb_ref[...], w_ref[...], preferred_element_type=jnp.float32)
    y = dis * xw
    ya_ref[...] = y[:, :HALF]
    yb_ref[...] = y[:, HALF:]


def _y_call(emb, w_gcn, d0, d1):
    blk = 2000
    grid = (N_NODES // blk,)
    return pl.pallas_call(
        _y_body,
        grid=grid,
        in_specs=[
            pl.BlockSpec((blk, IN_DIM), lambda i: (i, 0)),
            pl.BlockSpec((IN_DIM, HID_DIM), lambda i: (0, 0)),
            pl.BlockSpec((blk, 1), lambda i: (i, 0)),
            pl.BlockSpec((blk, 1), lambda i: (i, 0)),
        ],
        out_specs=[
            pl.BlockSpec((blk, HALF), lambda i: (i, 0)),
            pl.BlockSpec((blk, HALF), lambda i: (i, 0)),
        ],
        out_shape=[
            jax.ShapeDtypeStruct((N_NODES, HALF), jnp.float32),
            jax.ShapeDtypeStruct((N_NODES, HALF), jnp.float32),
        ],
    )(emb, w_gcn, d0, d1)


def _z_body(pa_ref, pb_ref, d0_ref, d1_ref, bg_ref, wo_ref, bo_ref, z_ref):
    deg = d0_ref[...] + d1_ref[...] + 1.0
    dis = lax.rsqrt(deg)
    h = dis * jnp.concatenate([pa_ref[...], pb_ref[...]], axis=1)
    e = jnp.maximum(h + bg_ref[...], 0.0)
    z_ref[...] = (
        jnp.dot(e, wo_ref[...], preferred_element_type=jnp.float32)
        + bo_ref[...]
    )


def _z_call(pa, pb, d0, d1, b_gcn, w_out_pad, b_out_pad, ncls_pad):
    blk = 2000
    grid = (N_NODES // blk,)
    half_spec = pl.BlockSpec((blk, HALF), lambda i: (i, 0))
    col_spec = pl.BlockSpec((blk, 1), lambda i: (i, 0))
    return pl.pallas_call(
        _z_body,
        grid=grid,
        in_specs=[
            half_spec, half_spec,
            col_spec, col_spec,
            pl.BlockSpec((1, HID_DIM), lambda i: (0, 0)),
            pl.BlockSpec((HID_DIM, ncls_pad), lambda i: (0, 0)),
            pl.BlockSpec((1, ncls_pad), lambda i: (0, 0)),
        ],
        out_specs=pl.BlockSpec((blk, ncls_pad), lambda i: (i, 0)),
        out_shape=jax.ShapeDtypeStruct((N_NODES, ncls_pad), jnp.float32),
    )(pa, pb, d0, d1, b_gcn, w_out_pad, b_out_pad)


def kernel(embedding, edge_index, W_gcn, b_gcn, W_out, b_out):
    src = edge_index[0].astype(jnp.int32)
    dst = edge_index[1].astype(jnp.int32)

    npad = E_PAD - N_EDGES
    pad_i = jnp.arange(npad, dtype=jnp.int32)
    pad_src = (pad_i * 131) % N_NODES
    pad_dst = N_NODES + (pad_i % PAD_ROWS)
    src_flat = jnp.concatenate([src, pad_src])
    dst_flat = jnp.concatenate([dst, pad_dst])
    dstp_deg = dst_flat.reshape(NW, NCHUNK, CHUNK)
    srca = src_flat.reshape(NS, NCHUNK2, CHUNK)
    srcb = srca + N_PAD
    dstp = dst_flat.reshape(NS, NCHUNK2, CHUNK)

    deg_pair = _deg_kernel(dstp_deg).reshape(NC, N_PAD)
    d0 = deg_pair[0, :N_NODES].reshape(N_NODES, 1)
    d1 = deg_pair[1, :N_NODES].reshape(N_NODES, 1)

    ya, yb = _y_call(embedding, W_gcn, d0, d1)
    zpad = jnp.zeros((N_PAD - N_NODES, HALF), jnp.float32)
    y2 = jnp.concatenate([ya, zpad, yb, zpad], axis=0)

    parts = _edge_kernel(y2, srca, srcb, dstp)

    ncls_pad = 128
    w_out_pad = jnp.zeros((HID_DIM, ncls_pad), jnp.float32).at[:, :NUM_CLASS].set(W_out)
    b_out_pad = jnp.zeros((1, ncls_pad), jnp.float32).at[0, :NUM_CLASS].set(b_out)

    z_pad = _z_call(parts[0, :N_NODES], parts[1, :N_NODES],
                    d0, d1, b_gcn.reshape(1, HID_DIM),
                    w_out_pad, b_out_pad, ncls_pad)
    return z_pad[:, :NUM_CLASS]


# speedup vs baseline: 41.0443x; 1.0078x over previous
import functools

import jax
import jax.numpy as jnp
from jax import lax
from jax.experimental import pallas as pl
from jax.experimental.pallas import tpu as pltpu, tpu_sc as plsc

N_NODES = 50000
IN_DIM = 64
HID_DIM = 32
HALF = HID_DIM // 2
NUM_CLASS = 10
N_EDGES = 800000

NC = 2
NS = 16
NW = NC * NS
L = 16

CHUNK = 128
EPT = 25088
NCHUNK = EPT // CHUNK
E_PAD = EPT * NW
N_PAD = 50176
STRIPE = N_PAD // NS
PAD_ROWS = N_PAD - N_NODES
BOUNCE = STRIPE // 4


def _deg_body(dst_hbm, deg_out, dst_v, ones_v, zero_v, deg_sh):
    c = lax.axis_index("c")
    s = lax.axis_index("s")
    w = c * NS + s

    for i in range(CHUNK // L):
        ones_v[pl.ds(i * L, L)] = jnp.ones((L,), jnp.float32)

    @pl.loop(0, STRIPE // L)
    def _zero(i):
        zero_v[pl.ds(i * L, L)] = jnp.zeros((L,), jnp.float32)

    pltpu.sync_copy(zero_v, deg_sh.at[pl.ds(s * STRIPE, STRIPE)])
    plsc.subcore_barrier()

    pltpu.sync_copy(dst_hbm.at[w], dst_v)

    @pl.loop(0, NCHUNK)
    def _scatter(j):
        pltpu.sync_copy(ones_v, deg_sh.at[dst_v.at[j]], add=True)

    plsc.subcore_barrier()
    pltpu.sync_copy(deg_sh.at[pl.ds(s * STRIPE, STRIPE)], zero_v)
    pltpu.sync_copy(zero_v, deg_out.at[pl.ds(c * N_PAD + s * STRIPE, STRIPE)])


_deg_kernel = functools.partial(
    pl.kernel,
    out_type=jax.ShapeDtypeStruct((NC * N_PAD,), jnp.float32),
    mesh=plsc.VectorSubcoreMesh(core_axis_name="c", subcore_axis_name="s"),
    compiler_params=pltpu.CompilerParams(use_tc_tiling_on_sc=False),
    scratch_types=[
        pltpu.VMEM((NCHUNK, CHUNK), jnp.int32),
        pltpu.VMEM((CHUNK,), jnp.float32),
        pltpu.VMEM((STRIPE,), jnp.float32),
        pltpu.VMEM_SHARED((N_PAD,), jnp.float32),
    ],
)(_deg_body)


NCHUNK2 = E_PAD // NS // CHUNK
SLAB = 56
NSLAB = NCHUNK2 // SLAB
NB = 8
LAG = 4


def _edge_body(y_hbm, src_hbm, dst_hbm, part_out,
               src_v, dst_v, r0, r1, r2, r3, r4, r5, r6, r7,
               g0, g1, g2, g3, g4, g5, g6, g7,
               s0, s1, s2, s3, s4, s5, s6, s7,
               bnc_v, h_sh):
    c = lax.axis_index("c")
    s = lax.axis_index("s")
    rows = [r0, r1, r2, r3, r4, r5, r6, r7]
    gsem = [g0, g1, g2, g3, g4, g5, g6, g7]
    ssem = [s0, s1, s2, s3, s4, s5, s6, s7]

    for k in range(4):
        r = s * STRIPE + k * BOUNCE
        pltpu.sync_copy(y_hbm.at[pl.ds(c * N_PAD + r, BOUNCE)], bnc_v)
        pltpu.sync_copy(bnc_v, h_sh.at[pl.ds(r, BOUNCE)])
    plsc.subcore_barrier()

    def g_issue(lj, b):
        pltpu.async_copy(y_hbm.at[src_v.at[lj]], rows[b], gsem[b])

    def g_wait(lj, b):
        pltpu.make_async_copy(y_hbm.at[src_v.at[lj]], rows[b], gsem[b]).wait()

    def s_issue(lj, b):
        pltpu.async_copy(rows[b], h_sh.at[dst_v.at[lj]], ssem[b], add=True)

    def s_wait(lj, b):
        pltpu.make_async_copy(rows[b], h_sh.at[dst_v.at[lj]], ssem[b]).wait()

    off = jnp.broadcast_to((c * N_PAD).astype(jnp.int32), (L,))

    @pl.loop(0, NSLAB)
    def _slab(t):
        pltpu.sync_copy(src_hbm.at[s, pl.ds(t * SLAB, SLAB)], src_v)
        pltpu.sync_copy(dst_hbm.at[s, pl.ds(t * SLAB, SLAB)], dst_v)

        @pl.loop(0, SLAB)
        def _off(i):
            for k in range(CHUNK // L):
                src_v[i, pl.ds(k * L, L)] = src_v[i, pl.ds(k * L, L)] + off

        for b in range(NB):
            g_issue(b, b)
        for b in range(LAG):
            g_wait(b, b)
            s_issue(b, b)

        @pl.loop(1, SLAB // NB)
        def _grp(g):
            for b in range(NB):
                lj = g * NB + b
                s_wait(lj - NB, b)
                g_issue(lj, b)
                bb = (b + LAG) % NB
                g_wait(lj - LAG, bb)
                s_issue(lj - LAG, bb)

        last = SLAB - NB
        for b in range(LAG):
            g_wait(last + LAG + b, LAG + b)
            s_issue(last + LAG + b, LAG + b)
        for b in range(NB):
            s_wait(last + b, b)

    plsc.subcore_barrier()
    for k in range(4):
        r = s * STRIPE + k * BOUNCE
        pltpu.sync_copy(h_sh.at[pl.ds(r, BOUNCE)], bnc_v)
        pltpu.sync_copy(bnc_v, part_out.at[c, pl.ds(r, BOUNCE)])


_edge_kernel = functools.partial(
    pl.kernel,
    out_type=jax.ShapeDtypeStruct((NC, N_PAD, HALF), jnp.float32),
    mesh=plsc.VectorSubcoreMesh(core_axis_name="c", subcore_axis_name="s"),
    compiler_params=pltpu.CompilerParams(use_tc_tiling_on_sc=False),
    scratch_types=(
        [
            pltpu.VMEM((SLAB, CHUNK), jnp.int32),
            pltpu.VMEM((SLAB, CHUNK), jnp.int32),
        ]
        + [pltpu.VMEM((CHUNK, HALF), jnp.float32) for _ in range(NB)]
        + [pltpu.SemaphoreType.DMA for _ in range(2 * NB)]
        + [
            pltpu.VMEM((BOUNCE, HALF), jnp.float32),
            pltpu.VMEM_SHARED((N_PAD, HALF), jnp.float32),
        ]
    ),
)(_edge_body)


def _y_body(emb_ref, w_ref, d0_ref, d1_ref, ya_ref, yb_ref):
    deg = d0_ref[...] + d1_ref[...] + 1.0
    dis = lax.rsqrt(deg)
    xw = jnp.dot(emb_ref[...], w_ref[...], preferred_element_type=jnp.float32)
    y = dis * xw
    ya_ref[...] = y[:, :HALF]
    yb_ref[...] = y[:, HALF:]


def _y_call(emb, w_gcn, d0, d1):
    blk = 10000
    grid = (N_NODES // blk,)
    return pl.pallas_call(
        _y_body,
        grid=grid,
        in_specs=[
            pl.BlockSpec((blk, IN_DIM), lambda i: (i, 0)),
            pl.BlockSpec((IN_DIM, HID_DIM), lambda i: (0, 0)),
            pl.BlockSpec((blk, 1), lambda i: (i, 0)),
            pl.BlockSpec((blk, 1), lambda i: (i, 0)),
        ],
        out_specs=[
            pl.BlockSpec((blk, HALF), lambda i: (i, 0)),
            pl.BlockSpec((blk, HALF), lambda i: (i, 0)),
        ],
        out_shape=[
            jax.ShapeDtypeStruct((N_NODES, HALF), jnp.float32),
            jax.ShapeDtypeStruct((N_NODES, HALF), jnp.float32),
        ],
    )(emb, w_gcn, d0, d1)


def _z_body(pa_ref, pb_ref, d0_ref, d1_ref, bg_ref, wo_ref, bo_ref, z_ref):
    deg = d0_ref[...] + d1_ref[...] + 1.0
    dis = lax.rsqrt(deg)
    h = dis * jnp.concatenate([pa_ref[...], pb_ref[...]], axis=1)
    e = jnp.maximum(h + bg_ref[...], 0.0)
    z_ref[...] = (
        jnp.dot(e, wo_ref[...], preferred_element_type=jnp.float32)
        + bo_ref[...]
    )


def _z_call(pa, pb, d0, d1, b_gcn, w_out_pad, b_out_pad, ncls_pad):
    blk = 10000
    grid = (N_NODES // blk,)
    half_spec = pl.BlockSpec((blk, HALF), lambda i: (i, 0))
    col_spec = pl.BlockSpec((blk, 1), lambda i: (i, 0))
    return pl.pallas_call(
        _z_body,
        grid=grid,
        in_specs=[
            half_spec, half_spec,
            col_spec, col_spec,
            pl.BlockSpec((1, HID_DIM), lambda i: (0, 0)),
            pl.BlockSpec((HID_DIM, ncls_pad), lambda i: (0, 0)),
            pl.BlockSpec((1, ncls_pad), lambda i: (0, 0)),
        ],
        out_specs=pl.BlockSpec((blk, ncls_pad), lambda i: (i, 0)),
        out_shape=jax.ShapeDtypeStruct((N_NODES, ncls_pad), jnp.float32),
    )(pa, pb, d0, d1, b_gcn, w_out_pad, b_out_pad)


def kernel(embedding, edge_index, W_gcn, b_gcn, W_out, b_out):
    src = edge_index[0].astype(jnp.int32)
    dst = edge_index[1].astype(jnp.int32)

    npad = E_PAD - N_EDGES
    pad_i = jnp.arange(npad, dtype=jnp.int32)
    pad_src = (pad_i * 131) % N_NODES
    pad_dst = N_NODES + (pad_i % PAD_ROWS)
    src_flat = jnp.concatenate([src, pad_src])
    dst_flat = jnp.concatenate([dst, pad_dst])
    dstp_deg = dst_flat.reshape(NW, NCHUNK, CHUNK)
    srcp = src_flat.reshape(NS, NCHUNK2, CHUNK)
    dstp = dst_flat.reshape(NS, NCHUNK2, CHUNK)

    deg_pair = _deg_kernel(dstp_deg).reshape(NC, N_PAD)
    d0 = deg_pair[0, :N_NODES].reshape(N_NODES, 1)
    d1 = deg_pair[1, :N_NODES].reshape(N_NODES, 1)

    ya, yb = _y_call(embedding, W_gcn, d0, d1)
    zpad = jnp.zeros((N_PAD - N_NODES, HALF), jnp.float32)
    y2 = jnp.concatenate([ya, zpad, yb, zpad], axis=0)

    parts = _edge_kernel(y2, srcp, dstp)

    ncls_pad = 16
    w_out_pad = jnp.zeros((HID_DIM, ncls_pad), jnp.float32).at[:, :NUM_CLASS].set(W_out)
    b_out_pad = jnp.zeros((1, ncls_pad), jnp.float32).at[0, :NUM_CLASS].set(b_out)

    z_pad = _z_call(parts[0, :N_NODES], parts[1, :N_NODES],
                    d0, d1, b_gcn.reshape(1, HID_DIM),
                    w_out_pad, b_out_pad, ncls_pad)
    return z_pad[:, :NUM_CLASS]


# speedup vs baseline: 54.6998x; 1.3327x over previous
import functools

import jax
import jax.numpy as jnp
from jax import lax
from jax.experimental import pallas as pl
from jax.experimental.pallas import tpu as pltpu, tpu_sc as plsc

N_NODES = 50000
IN_DIM = 64
HID_DIM = 32
HALF = HID_DIM // 2
NUM_CLASS = 10
N_EDGES = 800000

NC = 2
NS = 16
NW = NC * NS
L = 16

CHUNK = 128
N_CHUNKS = N_EDGES // CHUNK
MAIN_A = 192
MAIN_C = 384
N_MAIN = NW * MAIN_A
XTRA = 128
XTRA_A = XTRA // NW
XTRA_C = XTRA // NS
N_PAD = 50176
STRIPE = N_PAD // NS
PAD_ROWS = N_PAD - N_NODES
BOUNCE = STRIPE // 4
NPAD_E = XTRA * CHUNK - (N_EDGES - N_MAIN * CHUNK)


def _deg_body(edges_hbm, xtra_hbm, deg_out, dst_v, ones_v, zero_v, deg_sh):
    c = lax.axis_index("c")
    s = lax.axis_index("s")
    w = c * NS + s

    for i in range(CHUNK // L):
        ones_v[pl.ds(i * L, L)] = jnp.ones((L,), jnp.float32)

    @pl.loop(0, STRIPE // L)
    def _zero(i):
        zero_v[pl.ds(i * L, L)] = jnp.zeros((L,), jnp.float32)

    pltpu.sync_copy(zero_v, deg_sh.at[pl.ds(s * STRIPE, STRIPE)])
    plsc.subcore_barrier()

    pltpu.sync_copy(edges_hbm.at[1, pl.ds(w * MAIN_A, MAIN_A)],
                    dst_v.at[pl.ds(0, MAIN_A)])
    pltpu.sync_copy(xtra_hbm.at[1, pl.ds(w * XTRA_A, XTRA_A)],
                    dst_v.at[pl.ds(MAIN_A, XTRA_A)])

    @pl.loop(0, MAIN_A + XTRA_A)
    def _scatter(j):
        pltpu.sync_copy(ones_v, deg_sh.at[dst_v.at[j]], add=True)

    plsc.subcore_barrier()
    pltpu.sync_copy(deg_sh.at[pl.ds(s * STRIPE, STRIPE)], zero_v)
    pltpu.sync_copy(zero_v, deg_out.at[pl.ds(c * N_PAD + s * STRIPE, STRIPE)])


_deg_kernel = functools.partial(
    pl.kernel,
    out_type=jax.ShapeDtypeStruct((NC * N_PAD,), jnp.float32),
    mesh=plsc.VectorSubcoreMesh(core_axis_name="c", subcore_axis_name="s"),
    compiler_params=pltpu.CompilerParams(use_tc_tiling_on_sc=False),
    scratch_types=[
        pltpu.VMEM((MAIN_A + XTRA_A, CHUNK), jnp.int32),
        pltpu.VMEM((CHUNK,), jnp.float32),
        pltpu.VMEM((STRIPE,), jnp.float32),
        pltpu.VMEM_SHARED((N_PAD,), jnp.float32),
    ],
)(_deg_body)


SLAB = 48
NSLAB = MAIN_C // SLAB
NB = 8
LAG = 4


def _edge_body(y_hbm, edges_hbm, xtra_hbm, part_out,
               src_v, dst_v, r0, r1, r2, r3, r4, r5, r6, r7,
               g0, g1, g2, g3, g4, g5, g6, g7,
               s0, s1, s2, s3, s4, s5, s6, s7,
               bnc_v, h_sh):
    c = lax.axis_index("c")
    s = lax.axis_index("s")
    rows = [r0, r1, r2, r3, r4, r5, r6, r7]
    gsem = [g0, g1, g2, g3, g4, g5, g6, g7]
    ssem = [s0, s1, s2, s3, s4, s5, s6, s7]

    for k in range(4):
        r = s * STRIPE + k * BOUNCE
        pltpu.sync_copy(y_hbm.at[c, pl.ds(r, BOUNCE)], bnc_v)
        pltpu.sync_copy(bnc_v, h_sh.at[pl.ds(r, BOUNCE)])
    plsc.subcore_barrier()

    def g_issue(lj, b):
        pltpu.async_copy(y_hbm.at[c].at[src_v.at[lj]], rows[b], gsem[b])

    def g_wait(lj, b):
        pltpu.make_async_copy(
            y_hbm.at[c].at[src_v.at[lj]], rows[b], gsem[b]).wait()

    def s_issue(lj, b):
        pltpu.async_copy(rows[b], h_sh.at[dst_v.at[lj]], ssem[b], add=True)

    def s_wait(lj, b):
        pltpu.make_async_copy(rows[b], h_sh.at[dst_v.at[lj]], ssem[b]).wait()

    def ring(n):
        for b in range(NB):
            g_issue(b, b)
        for b in range(LAG):
            g_wait(b, b)
            s_issue(b, b)

        if n > NB:
            @pl.loop(1, n // NB)
            def _grp(g):
                for b in range(NB):
                    lj = g * NB + b
                    s_wait(lj - NB, b)
                    g_issue(lj, b)
                    bb = (b + LAG) % NB
                    g_wait(lj - LAG, bb)
                    s_issue(lj - LAG, bb)

        last = n - NB
        for b in range(LAG):
            g_wait(last + LAG + b, LAG + b)
            s_issue(last + LAG + b, LAG + b)
        for b in range(NB):
            s_wait(last + b, b)

    @pl.loop(0, NSLAB)
    def _slab(t):
        base = s * MAIN_C + t * SLAB
        pltpu.sync_copy(edges_hbm.at[0, pl.ds(base, SLAB)], src_v)
        pltpu.sync_copy(edges_hbm.at[1, pl.ds(base, SLAB)], dst_v)
        ring(SLAB)

    pltpu.sync_copy(xtra_hbm.at[0, pl.ds(s * XTRA_C, XTRA_C)],
                    src_v.at[pl.ds(0, XTRA_C)])
    pltpu.sync_copy(xtra_hbm.at[1, pl.ds(s * XTRA_C, XTRA_C)],
                    dst_v.at[pl.ds(0, XTRA_C)])
    ring(XTRA_C)

    plsc.subcore_barrier()
    for k in range(4):
        r = s * STRIPE + k * BOUNCE
        pltpu.sync_copy(h_sh.at[pl.ds(r, BOUNCE)], bnc_v)
        pltpu.sync_copy(bnc_v, part_out.at[c, pl.ds(r, BOUNCE)])


_edge_kernel = functools.partial(
    pl.kernel,
    out_type=jax.ShapeDtypeStruct((NC, N_PAD, HALF), jnp.float32),
    mesh=plsc.VectorSubcoreMesh(core_axis_name="c", subcore_axis_name="s"),
    compiler_params=pltpu.CompilerParams(use_tc_tiling_on_sc=False),
    scratch_types=(
        [
            pltpu.VMEM((SLAB, CHUNK), jnp.int32),
            pltpu.VMEM((SLAB, CHUNK), jnp.int32),
        ]
        + [pltpu.VMEM((CHUNK, HALF), jnp.float32) for _ in range(NB)]
        + [pltpu.SemaphoreType.DMA for _ in range(2 * NB)]
        + [
            pltpu.VMEM((BOUNCE, HALF), jnp.float32),
            pltpu.VMEM_SHARED((N_PAD, HALF), jnp.float32),
        ]
    ),
)(_edge_body)


def _y_body(emb_ref, w_ref, d0_ref, d1_ref, y_ref):
    deg = d0_ref[...] + d1_ref[...] + 1.0
    dis = lax.rsqrt(deg)
    xw = jnp.dot(emb_ref[...], w_ref[...], preferred_element_type=jnp.float32)
    y = dis * xw
    y_ref[0] = y[:, :HALF]
    y_ref[1] = y[:, HALF:]


def _y_call(emb, w_gcn, d0, d1):
    blk = 10000
    grid = (N_NODES // blk,)
    return pl.pallas_call(
        _y_body,
        grid=grid,
        in_specs=[
            pl.BlockSpec((blk, IN_DIM), lambda i: (i, 0)),
            pl.BlockSpec((IN_DIM, HID_DIM), lambda i: (0, 0)),
            pl.BlockSpec((blk, 1), lambda i: (i, 0)),
            pl.BlockSpec((blk, 1), lambda i: (i, 0)),
        ],
        out_specs=pl.BlockSpec((NC, blk, HALF), lambda i: (0, i, 0)),
        out_shape=jax.ShapeDtypeStruct((NC, N_PAD, HALF), jnp.float32),
    )(emb, w_gcn, d0, d1)


def _z_body(parts_ref, d0_ref, d1_ref, bg_ref, wo_ref, bo_ref, z_ref):
    deg = d0_ref[...] + d1_ref[...] + 1.0
    dis = lax.rsqrt(deg)
    h = dis * jnp.concatenate([parts_ref[0], parts_ref[1]], axis=1)
    e = jnp.maximum(h + bg_ref[...], 0.0)
    z = (
        jnp.dot(e, wo_ref[...], preferred_element_type=jnp.float32)
        + bo_ref[...]
    )
    z_ref[...] = z[:, :NUM_CLASS]


def _z_call(parts, d0, d1, b_gcn, w_out_pad, b_out_pad, ncls_pad):
    blk = 10000
    grid = (N_NODES // blk,)
    col_spec = pl.BlockSpec((blk, 1), lambda i: (i, 0))
    return pl.pallas_call(
        _z_body,
        grid=grid,
        in_specs=[
            pl.BlockSpec((NC, blk, HALF), lambda i: (0, i, 0)),
            col_spec, col_spec,
            pl.BlockSpec((1, HID_DIM), lambda i: (0, 0)),
            pl.BlockSpec((HID_DIM, ncls_pad), lambda i: (0, 0)),
            pl.BlockSpec((1, ncls_pad), lambda i: (0, 0)),
        ],
        out_specs=pl.BlockSpec((blk, NUM_CLASS), lambda i: (i, 0)),
        out_shape=jax.ShapeDtypeStruct((N_NODES, NUM_CLASS), jnp.float32),
    )(parts, d0, d1, b_gcn, w_out_pad, b_out_pad)


def kernel(embedding, edge_index, W_gcn, b_gcn, W_out, b_out):
    ei = edge_index.astype(jnp.int32)
    edges = ei.reshape(2, N_CHUNKS, CHUNK)

    pad_i = jnp.arange(NPAD_E, dtype=jnp.int32)
    pad_src = (pad_i * 131) % N_NODES
    pad_dst = N_NODES + (pad_i % PAD_ROWS)
    tail = ei[:, N_MAIN * CHUNK:]
    pads = jnp.stack([pad_src, pad_dst])
    xtra = jnp.concatenate([tail, pads], axis=1).reshape(2, XTRA, CHUNK)

    deg_pair = _deg_kernel(edges, xtra).reshape(NC, N_PAD)
    d0 = deg_pair[0, :N_NODES].reshape(N_NODES, 1)
    d1 = deg_pair[1, :N_NODES].reshape(N_NODES, 1)

    y2 = _y_call(embedding, W_gcn, d0, d1)
    parts = _edge_kernel(y2, edges, xtra)

    ncls_pad = 16
    w_out_pad = jnp.zeros((HID_DIM, ncls_pad), jnp.float32).at[:, :NUM_CLASS].set(W_out)
    b_out_pad = jnp.zeros((1, ncls_pad), jnp.float32).at[0, :NUM_CLASS].set(b_out)

    return _z_call(parts, d0, d1, b_gcn.reshape(1, HID_DIM),
                   w_out_pad, b_out_pad, ncls_pad)
